# baseline (speedup 1.0000x reference)
import functools

import jax
import jax.numpy as jnp
from jax import lax
from jax.experimental import pallas as pl
from jax.experimental.pallas import tpu as pltpu
from jax.experimental.pallas import tpu_sc as plsc

_LANES = 16
_CHUNK = 64
_NBUF = 4


def _emb_body(n_chunks, n_per_w, D, scale,
              idx_hbm, table_hbm, out_hbm, idx_v, gbufs, sbufs,
              isems, gsems, ssems):
    wid = lax.axis_index("s") * 2 + lax.axis_index("c")
    chunk_base = wid * n_chunks
    row_base = wid * n_per_w
    n_rounds = n_chunks // _NBUF
    scale_v = jnp.full((_LANES,), scale, dtype=jnp.float32)

    def start_idx(g, b):
        pltpu.async_copy(idx_hbm.at[chunk_base + g], idx_v[b], isems[b])

    def wait_idx(g, b):
        pltpu.make_async_copy(
            idx_hbm.at[chunk_base + g], idx_v[b], isems[b]).wait()

    def start_gather(g, b):
        pltpu.async_copy(table_hbm.at[idx_v[b]], gbufs[b], gsems[b])

    def wait_gather(g, b):
        pltpu.make_async_copy(
            table_hbm.at[idx_v[b]], gbufs[b], gsems[b]).wait()

    def out_slice(g):
        return out_hbm.at[pl.ds(row_base + g * _CHUNK, _CHUNK)]

    def start_scatter(g, b):
        pltpu.async_copy(sbufs[b], out_slice(g), ssems[b])

    def wait_scatter(g, b):
        pltpu.make_async_copy(sbufs[b], out_slice(g), ssems[b]).wait()

    def scale_chunk(b):
        def row_body(r, carry):
            for c in range(D // _LANES):
                sbufs[b][r, pl.ds(c * _LANES, _LANES)] = (
                    gbufs[b][r, pl.ds(c * _LANES, _LANES)] * scale_v)
            return carry

        lax.fori_loop(0, _CHUNK, row_body, 0, unroll=8)

    for b in range(_NBUF):
        start_idx(b, b)
        wait_idx(b, b)
        start_gather(b, b)

    def round_body(t, carry):
        for b in range(_NBUF):
            g = t * _NBUF + b
            wait_gather(g, b)

            @pl.when(t < n_rounds - 1)
            def _():
                start_idx(g + _NBUF, b)

            @pl.when(t > 0)
            def _():
                wait_scatter(g - _NBUF, b)

            scale_chunk(b)

            @pl.when(t < n_rounds - 1)
            def _():
                wait_idx(g + _NBUF, b)
                start_gather(g + _NBUF, b)

            start_scatter(g, b)
        return carry

    lax.fori_loop(0, n_rounds, round_body, 0)
    for b in range(_NBUF):
        wait_scatter((n_rounds - 1) * _NBUF + b, b)


def kernel(input_x, table):
    B0, S = input_x.shape
    V, D = table.shape
    B = B0 * S
    n_workers = 32
    n_per_w = B // n_workers
    n_chunks = n_per_w // _CHUNK
    scale = float(D) ** 0.5

    idx2d = input_x.reshape(B // _CHUNK, _CHUNK).astype(jnp.int32)
    table_pad = jnp.pad(table, ((0, 0), (0, 128 - D)))

    mesh = plsc.VectorSubcoreMesh(core_axis_name="c", subcore_axis_name="s")
    emb = pl.kernel(
        functools.partial(_emb_body, n_chunks, n_per_w, D, scale),
        mesh=mesh,
        out_type=jax.ShapeDtypeStruct((B, D), jnp.float32),
        scratch_types=[
            [pltpu.VMEM((_CHUNK,), jnp.int32) for _ in range(_NBUF)],
            [pltpu.VMEM((_CHUNK, 128), jnp.float32) for _ in range(_NBUF)],
            [pltpu.VMEM((_CHUNK, D), jnp.float32) for _ in range(_NBUF)],
            [pltpu.SemaphoreType.DMA for _ in range(_NBUF)],
            [pltpu.SemaphoreType.DMA for _ in range(_NBUF)],
            [pltpu.SemaphoreType.DMA for _ in range(_NBUF)],
        ],
        compiler_params=pltpu.CompilerParams(use_tc_tiling_on_sc=True),
    )
    out = emb(idx2d, table_pad)
    return out.reshape(B0, S, D)


# speedup vs baseline: 1.1662x; 1.1662x over previous
import functools

import jax
import jax.numpy as jnp
from jax import lax
from jax.experimental import pallas as pl
from jax.experimental.pallas import tpu as pltpu
from jax.experimental.pallas import tpu_sc as plsc

_LANES = 16
_CHUNK = 64
_NBUF = 4


def _emb_body(n_chunks, n_per_w, D, scale,
              idx_hbm, table_hbm, out_hbm, idx_v, gbufs, sbufs,
              isems, gsems, ssems):
    wid = lax.axis_index("s") * 2 + lax.axis_index("c")
    chunk_base = wid * n_chunks
    row_base = wid * n_per_w
    n_rounds = n_chunks // _NBUF
    scale_v = jnp.full((_LANES,), scale, dtype=jnp.float32)

    def start_idx(g, b):
        pltpu.async_copy(idx_hbm.at[chunk_base + g], idx_v[b], isems[b])

    def wait_idx(g, b):
        pltpu.make_async_copy(
            idx_hbm.at[chunk_base + g], idx_v[b], isems[b]).wait()

    def start_gather(g, b):
        pltpu.async_copy(table_hbm.at[idx_v[b]], gbufs[b], gsems[b])

    def wait_gather(g, b):
        pltpu.make_async_copy(
            table_hbm.at[idx_v[b]], gbufs[b], gsems[b]).wait()

    def out_slice(g):
        return out_hbm.at[pl.ds(row_base + g * _CHUNK, _CHUNK)]

    def start_scatter(g, b):
        pltpu.async_copy(sbufs[b], out_slice(g), ssems[b])

    def wait_scatter(g, b):
        pltpu.make_async_copy(sbufs[b], out_slice(g), ssems[b]).wait()

    def scale_chunk(b):
        @plsc.parallel_loop(0, _CHUNK, unroll=8)
        def _(r):
            for c in range(D // _LANES):
                sbufs[b][r, pl.ds(c * _LANES, _LANES)] = (
                    gbufs[b][r, pl.ds(c * _LANES, _LANES)] * scale_v)

    for b in range(_NBUF):
        start_idx(b, b)
        wait_idx(b, b)
        start_gather(b, b)

    def round_body(t, carry):
        for b in range(_NBUF):
            g = t * _NBUF + b
            wait_gather(g, b)

            @pl.when(t < n_rounds - 1)
            def _():
                start_idx(g + _NBUF, b)

            @pl.when(t > 0)
            def _():
                wait_scatter(g - _NBUF, b)

            scale_chunk(b)

            @pl.when(t < n_rounds - 1)
            def _():
                wait_idx(g + _NBUF, b)
                start_gather(g + _NBUF, b)

            start_scatter(g, b)
        return carry

    lax.fori_loop(0, n_rounds, round_body, 0)
    for b in range(_NBUF):
        wait_scatter((n_rounds - 1) * _NBUF + b, b)


def kernel(input_x, table):
    B0, S = input_x.shape
    V, D = table.shape
    B = B0 * S
    n_workers = 32
    n_per_w = B // n_workers
    n_chunks = n_per_w // _CHUNK
    scale = float(D) ** 0.5

    idx2d = input_x.reshape(B // _CHUNK, _CHUNK).astype(jnp.int32)
    table_pad = jnp.pad(table, ((0, 0), (0, 128 - D)))

    mesh = plsc.VectorSubcoreMesh(core_axis_name="c", subcore_axis_name="s")
    emb = pl.kernel(
        functools.partial(_emb_body, n_chunks, n_per_w, D, scale),
        mesh=mesh,
        out_type=jax.ShapeDtypeStruct((B, D), jnp.float32),
        scratch_types=[
            [pltpu.VMEM((_CHUNK,), jnp.int32) for _ in range(_NBUF)],
            [pltpu.VMEM((_CHUNK, 128), jnp.float32) for _ in range(_NBUF)],
            [pltpu.VMEM((_CHUNK, D), jnp.float32) for _ in range(_NBUF)],
            [pltpu.SemaphoreType.DMA for _ in range(_NBUF)],
            [pltpu.SemaphoreType.DMA for _ in range(_NBUF)],
            [pltpu.SemaphoreType.DMA for _ in range(_NBUF)],
        ],
        compiler_params=pltpu.CompilerParams(use_tc_tiling_on_sc=True),
    )
    out = emb(idx2d, table_pad)
    return out.reshape(B0, S, D)


# speedup vs baseline: 1.1738x; 1.0065x over previous
import functools

import jax
import jax.numpy as jnp
from jax import lax
from jax.experimental import pallas as pl
from jax.experimental.pallas import tpu as pltpu
from jax.experimental.pallas import tpu_sc as plsc

_LANES = 16
_CHUNK = 128
_NBUF = 2


def _emb_body(n_chunks, n_per_w, D, scale,
              idx_hbm, table_hbm, out_hbm, idx_v, gbufs, sbufs,
              isems, gsems, ssems):
    wid = lax.axis_index("s") * 2 + lax.axis_index("c")
    chunk_base = wid * n_chunks
    row_base = wid * n_per_w
    n_rounds = n_chunks // _NBUF
    scale_v = jnp.full((_LANES,), scale, dtype=jnp.float32)

    def start_idx(g, b):
        pltpu.async_copy(idx_hbm.at[chunk_base + g], idx_v[b], isems[b])

    def wait_idx(g, b):
        pltpu.make_async_copy(
            idx_hbm.at[chunk_base + g], idx_v[b], isems[b]).wait()

    def start_gather(g, b):
        pltpu.async_copy(table_hbm.at[idx_v[b]], gbufs[b], gsems[b])

    def wait_gather(g, b):
        pltpu.make_async_copy(
            table_hbm.at[idx_v[b]], gbufs[b], gsems[b]).wait()

    def out_slice(g):
        return out_hbm.at[pl.ds(row_base + g * _CHUNK, _CHUNK)]

    def start_scatter(g, b):
        pltpu.async_copy(sbufs[b], out_slice(g), ssems[b])

    def wait_scatter(g, b):
        pltpu.make_async_copy(sbufs[b], out_slice(g), ssems[b]).wait()

    def scale_chunk(b):
        @plsc.parallel_loop(0, _CHUNK, unroll=8)
        def _(r):
            for c in range(D // _LANES):
                sbufs[b][r, pl.ds(c * _LANES, _LANES)] = (
                    gbufs[b][r, pl.ds(c * _LANES, _LANES)] * scale_v)

    for b in range(_NBUF):
        start_idx(b, b)
        wait_idx(b, b)
        start_gather(b, b)

    def round_body(t, carry):
        for b in range(_NBUF):
            g = t * _NBUF + b
            wait_gather(g, b)

            @pl.when(t < n_rounds - 1)
            def _():
                start_idx(g + _NBUF, b)

            @pl.when(t > 0)
            def _():
                wait_scatter(g - _NBUF, b)

            scale_chunk(b)

            @pl.when(t < n_rounds - 1)
            def _():
                wait_idx(g + _NBUF, b)
                start_gather(g + _NBUF, b)

            start_scatter(g, b)
        return carry

    lax.fori_loop(0, n_rounds, round_body, 0)
    for b in range(_NBUF):
        wait_scatter((n_rounds - 1) * _NBUF + b, b)


def kernel(input_x, table):
    B0, S = input_x.shape
    V, D = table.shape
    B = B0 * S
    n_workers = 32
    n_per_w = B // n_workers
    n_chunks = n_per_w // _CHUNK
    scale = float(D) ** 0.5

    idx2d = input_x.reshape(B // _CHUNK, _CHUNK).astype(jnp.int32)
    table_pad = jnp.pad(table, ((0, 0), (0, 128 - D)))

    mesh = plsc.VectorSubcoreMesh(core_axis_name="c", subcore_axis_name="s")
    emb = pl.kernel(
        functools.partial(_emb_body, n_chunks, n_per_w, D, scale),
        mesh=mesh,
        out_type=jax.ShapeDtypeStruct((B, D), jnp.float32),
        scratch_types=[
            [pltpu.VMEM((_CHUNK,), jnp.int32) for _ in range(_NBUF)],
            [pltpu.VMEM((_CHUNK, 128), jnp.float32) for _ in range(_NBUF)],
            [pltpu.VMEM((_CHUNK, D), jnp.float32) for _ in range(_NBUF)],
            [pltpu.SemaphoreType.DMA for _ in range(_NBUF)],
            [pltpu.SemaphoreType.DMA for _ in range(_NBUF)],
            [pltpu.SemaphoreType.DMA for _ in range(_NBUF)],
        ],
        compiler_params=pltpu.CompilerParams(use_tc_tiling_on_sc=True),
    )
    out = emb(idx2d, table_pad)
    return out.reshape(B0, S, D)
